# baseline (speedup 1.0000x reference)
import functools

import jax
import jax.numpy as jnp
from jax import lax
from jax.experimental import pallas as pl
from jax.experimental.pallas import tpu as pltpu

_MAX_RANGE = 70
_DIM = 128
_TB = 512


def _tc_body(idx_ref, hi_ref, lo_ref, out_ref):
    idxb = idx_ref[0, 0, :]
    idxb = jnp.minimum(jnp.maximum(idxb, 0), _MAX_RANGE - 1)
    iota = lax.broadcasted_iota(jnp.int32, (_TB, _DIM), 1)
    oh = (idxb[:, None] == iota).astype(jnp.bfloat16)
    acc = jnp.dot(oh, hi_ref[...], preferred_element_type=jnp.float32)
    acc = acc + jnp.dot(oh, lo_ref[...], preferred_element_type=jnp.float32)
    out_ref[...] = acc


def _tc_gather(idx_flat, table):
    n = idx_flat.shape[0]
    grid = n // _TB
    idx3 = idx_flat.reshape(grid, 1, _TB)
    tpad = jnp.zeros((_DIM, _DIM), jnp.float32).at[:_MAX_RANGE].set(table)
    hi = tpad.astype(jnp.bfloat16)
    lo = (tpad - hi.astype(jnp.float32)).astype(jnp.bfloat16)
    return pl.pallas_call(
        _tc_body,
        grid=(grid,),
        in_specs=[
            pl.BlockSpec((1, 1, _TB), lambda i: (i, 0, 0)),
            pl.BlockSpec((_DIM, _DIM), lambda i: (0, 0)),
            pl.BlockSpec((_DIM, _DIM), lambda i: (0, 0)),
        ],
        out_specs=pl.BlockSpec((_TB, _DIM), lambda i: (i, 0)),
        out_shape=jax.ShapeDtypeStruct((n, _DIM), jnp.float32),
    )(idx3, hi, lo)


def kernel(prior_info, embedding):
    batch, hist = prior_info.shape
    idx_flat = prior_info.reshape(-1).astype(jnp.int32)
    out = _tc_gather(idx_flat, embedding)
    return out.reshape(batch, hist, _DIM)


# speedup vs baseline: 3.5624x; 3.5624x over previous
import functools

import jax
import jax.numpy as jnp
from jax import lax
from jax.experimental import pallas as pl
from jax.experimental.pallas import tpu as pltpu
from jax.experimental.pallas import tpu_sc as plsc

_MAX_RANGE = 70
_DIM = 128

_NC = 2
_NS = 16
_NW = _NC * _NS
_LANES = 16

_CHUNK = 128
_NBUF = 4


def _sc_gather(idx2d, table):
    n_rows = idx2d.shape[0]
    b_total = n_rows * _CHUNK
    rows_per_w = n_rows // _NW
    n_outer = rows_per_w // _NBUF
    mesh = plsc.VectorSubcoreMesh(core_axis_name="c", subcore_axis_name="s")

    @functools.partial(
        pl.kernel,
        mesh=mesh,
        out_type=jax.ShapeDtypeStruct((b_total, _DIM), jnp.float32),
        scratch_types=[
            pltpu.VMEM_SHARED((_MAX_RANGE, _DIM), jnp.float32),
            pltpu.VMEM((2, _NBUF, _CHUNK), jnp.int32),
            pltpu.VMEM((_NBUF, _CHUNK, _DIM), jnp.float32),
            pltpu.SemaphoreType.DMA,
            pltpu.SemaphoreType.DMA,
            pltpu.SemaphoreType.DMA,
        ] + [pltpu.SemaphoreType.DMA for _ in range(_NBUF)],
    )
    def k(table_hbm, idx_hbm, out_hbm, table_sh, idx_v, rows_v,
          sem_g, sem_i0, sem_i1, *sem_o):
        sem_i = (sem_i0, sem_i1)
        wid = lax.axis_index("s") * _NC + lax.axis_index("c")
        row0 = wid * rows_per_w

        @pl.when(lax.axis_index("s") == 0)
        def _stage_table():
            pltpu.sync_copy(table_hbm, table_sh)

        plsc.subcore_barrier()

        def idx_load(g, ph):
            return pltpu.make_async_copy(
                idx_hbm.at[pl.ds(row0 + g * _NBUF, _NBUF)],
                idx_v.at[ph], sem_i[ph])

        def out_copy(b, cbase):
            return pltpu.make_async_copy(
                rows_v.at[b], out_hbm.at[pl.ds(cbase, _CHUNK)], sem_o[b])

        def gather(ph, b):
            return pltpu.make_async_copy(
                table_sh.at[idx_v.at[ph].at[b]], rows_v.at[b], sem_g)

        idx_load(0, 0).start()

        def body(go, carry):
            for ph in range(2):
                g = go * 2 + ph

                @pl.when(g + 1 < n_outer)
                def _prefetch():
                    idx_load(g + 1, 1 - ph).start()

                idx_load(g, ph).wait()
                grow = row0 + g * _NBUF
                for b in range(_NBUF):
                    cbase = (grow + b) * _CHUNK

                    @pl.when(g >= 1)
                    def _wait_prev():
                        out_copy(b, cbase).wait()

                    for i in range(_CHUNK // _LANES):
                        sl = pl.ds(i * _LANES, _LANES)
                        v = idx_v[ph, b, sl]
                        idx_v[ph, b, sl] = jnp.minimum(
                            jnp.maximum(v, 0), _MAX_RANGE - 1)
                    gather(ph, b).start()
                for b in range(_NBUF):
                    gather(ph, b).wait()
                for b in range(_NBUF):
                    out_copy(b, (grow + b) * _CHUNK).start()
            return carry

        lax.fori_loop(0, n_outer // 2, body, 0)
        for b in range(_NBUF):
            out_copy(b, row0 * _CHUNK).wait()

    return k(table, idx2d)


def kernel(prior_info, embedding):
    batch, hist = prior_info.shape
    idx2d = prior_info.reshape(-1, _CHUNK).astype(jnp.int32)
    out = _sc_gather(idx2d, embedding)
    return out.reshape(batch, hist, _DIM)


# speedup vs baseline: 5.4097x; 1.5186x over previous
import functools

import jax
import jax.numpy as jnp
from jax import lax
from jax.experimental import pallas as pl
from jax.experimental.pallas import tpu as pltpu
from jax.experimental.pallas import tpu_sc as plsc

_MAX_RANGE = 70
_DIM = 128

_NC = 2
_NS = 16
_NW = _NC * _NS
_LANES = 16

_CHUNK = 128
_NBUF = 4


def _sc_gather(idx2d, table):
    n_rows = idx2d.shape[0]
    b_total = n_rows * _CHUNK
    rows_per_w = n_rows // _NW
    n_outer = rows_per_w // _NBUF
    mesh = plsc.VectorSubcoreMesh(core_axis_name="c", subcore_axis_name="s")

    @functools.partial(
        pl.kernel,
        mesh=mesh,
        out_type=jax.ShapeDtypeStruct((b_total, _DIM), jnp.float32),
        scratch_types=[
            pltpu.VMEM_SHARED((_MAX_RANGE, _DIM), jnp.float32),
            pltpu.VMEM((2, _NBUF, _CHUNK), jnp.int32),
            pltpu.VMEM((_NBUF, _CHUNK, _DIM), jnp.float32),
            pltpu.SemaphoreType.DMA,
            pltpu.SemaphoreType.DMA,
        ] + [pltpu.SemaphoreType.DMA for _ in range(2 * _NBUF)],
    )
    def k(table_hbm, idx_hbm, out_hbm, table_sh, idx_v, rows_v,
          sem_i0, sem_i1, *sems):
        sem_i = (sem_i0, sem_i1)
        sem_g = sems[:_NBUF]
        sem_o = sems[_NBUF:]
        wid = lax.axis_index("s") * _NC + lax.axis_index("c")
        row0 = wid * rows_per_w

        @pl.when(lax.axis_index("s") == 0)
        def _stage_table():
            pltpu.sync_copy(table_hbm, table_sh)

        plsc.subcore_barrier()

        def idx_load(g, ph):
            return pltpu.make_async_copy(
                idx_hbm.at[pl.ds(row0 + g * _NBUF, _NBUF)],
                idx_v.at[ph], sem_i[ph])

        def out_copy(b, cbase):
            return pltpu.make_async_copy(
                rows_v.at[b], out_hbm.at[pl.ds(cbase, _CHUNK)], sem_o[b])

        def gather(ph, b):
            return pltpu.make_async_copy(
                table_sh.at[idx_v.at[ph].at[b]], rows_v.at[b], sem_g[b])

        idx_load(0, 0).start()

        def body(go, carry):
            for ph in range(2):
                g = go * 2 + ph

                @pl.when(g + 1 < n_outer)
                def _prefetch():
                    idx_load(g + 1, 1 - ph).start()

                idx_load(g, ph).wait()
                grow = row0 + g * _NBUF
                for b in range(_NBUF):
                    for i in range(_CHUNK // _LANES):
                        sl = pl.ds(i * _LANES, _LANES)
                        v = idx_v[ph, b, sl]
                        idx_v[ph, b, sl] = jnp.minimum(
                            jnp.maximum(v, 0), _MAX_RANGE - 1)
                for b in range(_NBUF):
                    cbase = (grow + b) * _CHUNK

                    @pl.when(g >= 1)
                    def _wait_prev():
                        out_copy(b, cbase).wait()

                    gather(ph, b).start()
                for b in range(_NBUF):
                    gather(ph, b).wait()
                    out_copy(b, (grow + b) * _CHUNK).start()
            return carry

        lax.fori_loop(0, n_outer // 2, body, 0)
        for b in range(_NBUF):
            out_copy(b, row0 * _CHUNK).wait()

    return k(table, idx2d)


def kernel(prior_info, embedding):
    batch, hist = prior_info.shape
    idx2d = prior_info.reshape(-1, _CHUNK).astype(jnp.int32)
    out = _sc_gather(idx2d, embedding)
    return out.reshape(batch, hist, _DIM)


# speedup vs baseline: 5.4198x; 1.0019x over previous
import functools

import jax
import jax.numpy as jnp
from jax import lax
from jax.experimental import pallas as pl
from jax.experimental.pallas import tpu as pltpu
from jax.experimental.pallas import tpu_sc as plsc

_MAX_RANGE = 70
_DIM = 128

_NC = 2
_NS = 16
_NW = _NC * _NS
_LANES = 16

_CHUNK = 128
_NBUF = 4
_NHALF = 2
_HROWS = (_NBUF // _NHALF) * _CHUNK


def _sc_gather(idx2d, table):
    n_rows = idx2d.shape[0]
    b_total = n_rows * _CHUNK
    rows_per_w = n_rows // _NW
    n_outer = rows_per_w // _NBUF
    mesh = plsc.VectorSubcoreMesh(core_axis_name="c", subcore_axis_name="s")

    @functools.partial(
        pl.kernel,
        mesh=mesh,
        out_type=jax.ShapeDtypeStruct((b_total, _DIM), jnp.float32),
        scratch_types=[
            pltpu.VMEM_SHARED((_MAX_RANGE, _DIM), jnp.float32),
            pltpu.VMEM((2, _NBUF, _CHUNK), jnp.int32),
            pltpu.VMEM((_NBUF * _CHUNK, _DIM), jnp.float32),
            pltpu.SemaphoreType.DMA,
            pltpu.SemaphoreType.DMA,
        ] + [pltpu.SemaphoreType.DMA for _ in range(_NBUF + _NHALF)],
    )
    def k(table_hbm, idx_hbm, out_hbm, table_sh, idx_v, rows_v,
          sem_i0, sem_i1, *sems):
        sem_i = (sem_i0, sem_i1)
        sem_g = sems[:_NBUF]
        sem_o = sems[_NBUF:]
        wid = lax.axis_index("s") * _NC + lax.axis_index("c")
        row0 = wid * rows_per_w

        @pl.when(lax.axis_index("s") == 0)
        def _stage_table():
            pltpu.sync_copy(table_hbm, table_sh)

        plsc.subcore_barrier()

        def idx_load(g, ph):
            return pltpu.make_async_copy(
                idx_hbm.at[pl.ds(row0 + g * _NBUF, _NBUF)],
                idx_v.at[ph], sem_i[ph])

        def out_copy(h, cbase):
            return pltpu.make_async_copy(
                rows_v.at[pl.ds(h * _HROWS, _HROWS)],
                out_hbm.at[pl.ds(cbase, _HROWS)], sem_o[h])

        def gather(ph, b):
            return pltpu.make_async_copy(
                table_sh.at[idx_v.at[ph].at[b]],
                rows_v.at[pl.ds(b * _CHUNK, _CHUNK)], sem_g[b])

        idx_load(0, 0).start()

        def body(go, carry):
            for ph in range(2):
                g = go * 2 + ph

                @pl.when(g + 1 < n_outer)
                def _prefetch():
                    idx_load(g + 1, 1 - ph).start()

                idx_load(g, ph).wait()
                grow = row0 + g * _NBUF
                for b in range(_NBUF):
                    for i in range(_CHUNK // _LANES):
                        sl = pl.ds(i * _LANES, _LANES)
                        v = idx_v[ph, b, sl]
                        idx_v[ph, b, sl] = jnp.minimum(
                            jnp.maximum(v, 0), _MAX_RANGE - 1)
                for b in range(_NBUF):
                    cbase = grow * _CHUNK
                    if b % (_NBUF // _NHALF) == 0:
                        @pl.when(g >= 1)
                        def _wait_prev(b=b, cbase=cbase):
                            out_copy(b // (_NBUF // _NHALF), cbase).wait()

                    gather(ph, b).start()
                for h in range(_NHALF):
                    for b in range(h * (_NBUF // _NHALF),
                                   (h + 1) * (_NBUF // _NHALF)):
                        gather(ph, b).wait()
                    out_copy(h, grow * _CHUNK + h * _HROWS).start()
            return carry

        lax.fori_loop(0, n_outer // 2, body, 0)
        for h in range(_NHALF):
            out_copy(h, row0 * _CHUNK).wait()

    return k(table, idx2d)


def kernel(prior_info, embedding):
    batch, hist = prior_info.shape
    idx2d = prior_info.reshape(-1, _CHUNK).astype(jnp.int32)
    out = _sc_gather(idx2d, embedding)
    return out.reshape(batch, hist, _DIM)
